# baseline (speedup 1.0000x reference)
import functools

import jax
import jax.numpy as jnp
from jax.experimental import pallas as pl


def _matmul_block2(xa_ref, xb_ref, w_ref, o_ref):
    h = xa_ref.shape[0]
    o_ref[:h, :] = jax.lax.dot_general(
        xa_ref[...], w_ref[...],
        dimension_numbers=(((1,), (1,)), ((), ())),
        preferred_element_type=jnp.float32,
    )
    o_ref[h:, :] = jax.lax.dot_general(
        xb_ref[...], w_ref[...],
        dimension_numbers=(((1,), (1,)), ((), ())),
        preferred_element_type=jnp.float32,
    )


@functools.partial(jax.jit, static_argnames=("bm", "interpret"))
def _router_logits(hidden_states, W, bm=1024, interpret=False):
    T, K = hidden_states.shape
    N = W.shape[0]
    half = bm // 2
    return pl.pallas_call(
        _matmul_block2,
        grid=(T // bm,),
        in_specs=[
            pl.BlockSpec((half, K), lambda i: (2 * i, 0)),
            pl.BlockSpec((half, K), lambda i: (2 * i + 1, 0)),
            pl.BlockSpec((N, K), lambda i: (0, 0)),
        ],
        out_specs=pl.BlockSpec((bm, N), lambda i: (i, 0)),
        out_shape=jax.ShapeDtypeStruct((T, N), jnp.float32),
        interpret=interpret,
    )(hidden_states, hidden_states, W)


def kernel(hidden_states, W):
    return _router_logits(hidden_states, W)


# speedup vs baseline: 1.0072x; 1.0072x over previous
import functools

import jax
import jax.numpy as jnp
from jax.experimental import pallas as pl


def _matmul_block(x_ref, w0_ref, w1_ref, o_ref):
    k = pl.program_id(1)

    @pl.when(k == 0)
    def _init():
        o_ref[...] = jax.lax.dot_general(
            x_ref[...], w0_ref[...],
            dimension_numbers=(((1,), (1,)), ((), ())),
            preferred_element_type=jnp.float32,
        )

    @pl.when(k == 1)
    def _accum():
        o_ref[...] += jax.lax.dot_general(
            x_ref[...], w1_ref[...],
            dimension_numbers=(((1,), (1,)), ((), ())),
            preferred_element_type=jnp.float32,
        )


@functools.partial(jax.jit, static_argnames=("bm",))
def _router_logits(hidden_states, W, bm=1024):
    T, K = hidden_states.shape
    N = W.shape[0]
    kh = K // 2
    return pl.pallas_call(
        _matmul_block,
        grid=(T // bm, 2),
        in_specs=[
            pl.BlockSpec((bm, kh), lambda i, k: (i, k)),
            pl.BlockSpec((N, kh), lambda i, k: (0, 0)),
            pl.BlockSpec((N, kh), lambda i, k: (0, 1)),
        ],
        out_specs=pl.BlockSpec((bm, N), lambda i, k: (i, 0)),
        out_shape=jax.ShapeDtypeStruct((T, N), jnp.float32),
    )(hidden_states, W, W)


def kernel(hidden_states, W):
    return _router_logits(hidden_states, W)
